# Initial kernel scaffold; baseline (speedup 1.0000x reference)
import jax
import jax.numpy as jnp
from jax.experimental import pallas as pl


def kernel(x, edge_index, W0, b0, Wc1, bc1, Wc2, bc2, W1, b1):
    raise NotImplementedError("write your pallas kernel here")


# speedup vs baseline: 6.6074x; 6.6074x over previous
import functools

import jax
import jax.numpy as jnp
from jax import lax
from jax.experimental import pallas as pl
from jax.experimental.pallas import tpu as pltpu
from jax.experimental.pallas import tpu_sc as plsc

N_NODES = 10000
N_EDGES = 160000
D = 256
H = 128
NC = 2
NS = 16
CHUNK = 128
ACC_ROWS = 10240
ROWS_PER_TILE = ACC_ROWS // NS
E_PAD = NS * 80 * CHUNK
CPT = E_PAD // NS // CHUNK
CPW = E_PAD // (NC * NS) // CHUNK

_mesh = plsc.VectorSubcoreMesh(core_axis_name="c", subcore_axis_name="s",
                               num_cores=NC, num_subcores=NS)


def _deg_body(dst_h, ones_h, zz_h, deg_h, dacc, dstv, onesv, rows):
    c = lax.axis_index("c")
    s = lax.axis_index("s")
    w = c * NS + s
    pltpu.sync_copy(dst_h.at[w], dstv)
    pltpu.sync_copy(ones_h, onesv)
    pltpu.sync_copy(zz_h, rows)
    base = s * ROWS_PER_TILE
    for k in range(ROWS_PER_TILE // CHUNK):
        pltpu.sync_copy(rows, dacc.at[pl.ds(base + k * CHUNK, CHUNK)])
    plsc.subcore_barrier()

    def chunk(j, carry):
        pltpu.sync_copy(onesv, dacc.at[dstv.at[j]], add=True)
        return carry

    lax.fori_loop(0, CPW, chunk, 0)
    plsc.subcore_barrier()
    for k in range(ROWS_PER_TILE // CHUNK):
        pltpu.sync_copy(dacc.at[pl.ds(base + k * CHUNK, CHUNK)], rows)
        pltpu.sync_copy(rows, deg_h.at[c].at[pl.ds(base + k * CHUNK, CHUNK)])


_deg_kernel = functools.partial(
    pl.kernel,
    out_type=jax.ShapeDtypeStruct((NC, ACC_ROWS, H), jnp.float32),
    mesh=_mesh,
    scratch_types=[
        pltpu.VMEM_SHARED((ACC_ROWS, H), jnp.float32),
        pltpu.VMEM((CPW, CHUNK), jnp.int32),
        pltpu.VMEM((CHUNK, H), jnp.float32),
        pltpu.VMEM((CHUNK, H), jnp.float32),
    ],
)(_deg_body)


def _scat_body(src_h, dst_h, ylo_h, yhi_h, zz_h, alo_h, ahi_h,
               acc, srcv, dstv, rows):
    c = lax.axis_index("c")
    s = lax.axis_index("s")
    pltpu.sync_copy(src_h.at[s], srcv)
    pltpu.sync_copy(dst_h.at[s], dstv)
    pltpu.sync_copy(zz_h, rows)
    base = s * ROWS_PER_TILE
    for k in range(ROWS_PER_TILE // CHUNK):
        pltpu.sync_copy(rows, acc.at[pl.ds(base + k * CHUNK, CHUNK)])
    plsc.subcore_barrier()

    def run(table_h, out_h):
        def chunk(j, carry):
            pltpu.sync_copy(table_h.at[srcv.at[j]], rows)
            pltpu.sync_copy(rows, acc.at[dstv.at[j]], add=True)
            return carry

        lax.fori_loop(0, CPT, chunk, 0)
        plsc.subcore_barrier()
        for k in range(ROWS_PER_TILE // CHUNK):
            pltpu.sync_copy(acc.at[pl.ds(base + k * CHUNK, CHUNK)], rows)
            pltpu.sync_copy(rows, out_h.at[pl.ds(base + k * CHUNK, CHUNK)])

    @pl.when(c == 0)
    def _():
        run(ylo_h, alo_h)

    @pl.when(c == 1)
    def _():
        run(yhi_h, ahi_h)


_scat_kernel = functools.partial(
    pl.kernel,
    out_type=(jax.ShapeDtypeStruct((ACC_ROWS, H), jnp.float32),
              jax.ShapeDtypeStruct((ACC_ROWS, H), jnp.float32)),
    mesh=_mesh,
    scratch_types=[
        pltpu.VMEM_SHARED((ACC_ROWS, H), jnp.float32),
        pltpu.VMEM((CPT, CHUNK), jnp.int32),
        pltpu.VMEM((CPT, CHUNK), jnp.int32),
        pltpu.VMEM((CHUNK, H), jnp.float32),
    ],
)(_scat_body)


RM = 1000
GRID = N_NODES // RM


def _dis_from(d0, d1):
    deg = d0[:, :1] + d1[:, :1] + 1.0
    return lax.rsqrt(deg)


def _mmT(a, w):
    return lax.dot_general(a, w, (((1,), (1,)), ((), ())),
                           preferred_element_type=jnp.float32)


def _tc1_body(x_ref, w0_ref, b0_ref, wc_ref, d0_ref, d1_ref, ylo_ref, yhi_ref):
    dis = _dis_from(d0_ref[...], d1_ref[...])
    h = jnp.maximum(_mmT(x_ref[...], w0_ref[...]) + b0_ref[...], 0.0)
    y = _mmT(h, wc_ref[...]) * dis
    ylo_ref[...] = y[:, :H]
    yhi_ref[...] = y[:, H:]


def _tc2_body(alo_ref, ahi_ref, ylo_ref, yhi_ref, b_ref, wc_ref,
              d0_ref, d1_ref, olo_ref, ohi_ref):
    dis = _dis_from(d0_ref[...], d1_ref[...])
    a = jnp.concatenate([alo_ref[...], ahi_ref[...]], axis=1)
    y = jnp.concatenate([ylo_ref[...], yhi_ref[...]], axis=1)
    h = jnp.maximum(dis * (a + y) + b_ref[...], 0.0)
    y2 = _mmT(h, wc_ref[...]) * dis
    olo_ref[...] = y2[:, :H]
    ohi_ref[...] = y2[:, H:]


def _tc3_body(alo_ref, ahi_ref, ylo_ref, yhi_ref, b_ref, w1_ref, b1_ref,
              d0_ref, d1_ref, out_ref):
    dis = _dis_from(d0_ref[...], d1_ref[...])
    a = jnp.concatenate([alo_ref[...], ahi_ref[...]], axis=1)
    y = jnp.concatenate([ylo_ref[...], yhi_ref[...]], axis=1)
    h = jnp.maximum(dis * (a + y) + b_ref[...], 0.0)
    out_ref[...] = _mmT(h, w1_ref[...]) + b1_ref[0, 0]


def _rows_spec(w):
    return pl.BlockSpec((RM, w), lambda i: (i, 0))


def _full_spec(shape):
    return pl.BlockSpec(shape, lambda i: tuple(0 for _ in shape))


_tc1 = pl.pallas_call(
    _tc1_body,
    grid=(GRID,),
    in_specs=[_rows_spec(D), _full_spec((D, D)), _full_spec((1, D)),
              _full_spec((D, D)), _rows_spec(H), _rows_spec(H)],
    out_specs=(_rows_spec(H), _rows_spec(H)),
    out_shape=(jax.ShapeDtypeStruct((N_NODES, H), jnp.float32),
               jax.ShapeDtypeStruct((N_NODES, H), jnp.float32)),
)

_tc2 = pl.pallas_call(
    _tc2_body,
    grid=(GRID,),
    in_specs=[_rows_spec(H), _rows_spec(H), _rows_spec(H), _rows_spec(H),
              _full_spec((1, D)), _full_spec((D, D)),
              _rows_spec(H), _rows_spec(H)],
    out_specs=(_rows_spec(H), _rows_spec(H)),
    out_shape=(jax.ShapeDtypeStruct((N_NODES, H), jnp.float32),
               jax.ShapeDtypeStruct((N_NODES, H), jnp.float32)),
)

_tc3 = pl.pallas_call(
    _tc3_body,
    grid=(GRID,),
    in_specs=[_rows_spec(H), _rows_spec(H), _rows_spec(H), _rows_spec(H),
              _full_spec((1, D)), _full_spec((H, D)), _full_spec((1, 1)),
              _rows_spec(H), _rows_spec(H)],
    out_specs=pl.BlockSpec((RM, H), lambda i: (i, 0)),
    out_shape=jax.ShapeDtypeStruct((N_NODES, H), jnp.float32),
)


@jax.jit
def kernel(x, edge_index, W0, b0, Wc1, bc1, Wc2, bc2, W1, b1):
    src = edge_index[0].astype(jnp.int32)
    dst = edge_index[1].astype(jnp.int32)
    pad = E_PAD - N_EDGES
    srcp = jnp.concatenate([src, jnp.zeros((pad,), jnp.int32)])
    dstp = jnp.concatenate([dst, jnp.full((pad,), N_NODES, jnp.int32)])
    src3 = srcp.reshape(NS, CPT, CHUNK)
    dst3 = dstp.reshape(NS, CPT, CHUNK)
    dst3w = dstp.reshape(NC * NS, CPW, CHUNK)
    ones_h = jnp.ones((CHUNK, H), jnp.float32)
    zz_h = jnp.zeros((CHUNK, H), jnp.float32)

    deg2 = _deg_kernel(dst3w, ones_h, zz_h)
    d0, d1 = deg2[0], deg2[1]

    b0r = b0.reshape(1, D)
    bc1r = bc1.reshape(1, D)
    bc2r = bc2.reshape(1, D)
    b1r = b1.reshape(1, 1)

    y1lo, y1hi = _tc1(x, W0, b0r, Wc1, d0[:N_NODES], d1[:N_NODES])
    a1lo, a1hi = _scat_kernel(src3, dst3, y1lo, y1hi, zz_h)
    W1p = jnp.zeros((H, D), jnp.float32).at[0].set(W1[0])
    y2lo, y2hi = _tc2(a1lo[:N_NODES], a1hi[:N_NODES], y1lo, y1hi,
                      bc1r, Wc2, d0[:N_NODES], d1[:N_NODES])
    a2lo, a2hi = _scat_kernel(src3, dst3, y2lo, y2hi, zz_h)
    outp = _tc3(a2lo[:N_NODES], a2hi[:N_NODES], y2lo, y2hi,
                bc2r, W1p, b1r, d0[:N_NODES], d1[:N_NODES])
    return outp[:, :1]


# speedup vs baseline: 7.2365x; 1.0952x over previous
import functools

import jax
import jax.numpy as jnp
from jax import lax
from jax.experimental import pallas as pl
from jax.experimental.pallas import tpu as pltpu
from jax.experimental.pallas import tpu_sc as plsc

N_NODES = 10000
N_EDGES = 160000
D = 256
H = 128
NC = 2
NS = 16
CHUNK = 128
ACC_ROWS = 10240
ROWS_PER_TILE = ACC_ROWS // NS
E_PAD = NS * 80 * CHUNK
CPT = E_PAD // NS // CHUNK
CPW = E_PAD // (NC * NS) // CHUNK

_mesh = plsc.VectorSubcoreMesh(core_axis_name="c", subcore_axis_name="s",
                               num_cores=NC, num_subcores=NS)


def _deg_body(dst_h, ones_h, zz_h, deg_h, dacc, dstv, onesv, rows):
    c = lax.axis_index("c")
    s = lax.axis_index("s")
    w = c * NS + s
    pltpu.sync_copy(dst_h.at[w], dstv)
    pltpu.sync_copy(ones_h, onesv)
    pltpu.sync_copy(zz_h, rows)
    base = s * ROWS_PER_TILE
    for k in range(ROWS_PER_TILE // CHUNK):
        pltpu.sync_copy(rows, dacc.at[pl.ds(base + k * CHUNK, CHUNK)])
    plsc.subcore_barrier()

    def chunk(j, carry):
        pltpu.sync_copy(onesv, dacc.at[dstv.at[j]], add=True)
        return carry

    lax.fori_loop(0, CPW, chunk, 0)
    plsc.subcore_barrier()
    for k in range(ROWS_PER_TILE // CHUNK):
        pltpu.sync_copy(dacc.at[pl.ds(base + k * CHUNK, CHUNK)], rows)
        pltpu.sync_copy(rows, deg_h.at[c].at[pl.ds(base + k * CHUNK, CHUNK)])


_deg_kernel = functools.partial(
    pl.kernel,
    out_type=jax.ShapeDtypeStruct((NC, ACC_ROWS, H), jnp.float32),
    mesh=_mesh,
    scratch_types=[
        pltpu.VMEM_SHARED((ACC_ROWS, H), jnp.float32),
        pltpu.VMEM((CPW, CHUNK), jnp.int32),
        pltpu.VMEM((CHUNK, H), jnp.float32),
        pltpu.VMEM((CHUNK, H), jnp.float32),
    ],
)(_deg_body)


NBUF = 2
IBLK = 40


def _scat_body(src_h, dst_h, ylo_h, yhi_h, zz_h, alo_h, ahi_h,
               acc, sidx, didx, r0, r1, g0, g1, s0, s1):
    rows = (r0, r1)
    gsem = (g0, g1)
    ssem = (s0, s1)
    c = lax.axis_index("c")
    s = lax.axis_index("s")
    pltpu.sync_copy(zz_h, r0)
    base = s * ROWS_PER_TILE
    for k in range(ROWS_PER_TILE // CHUNK):
        pltpu.sync_copy(r0, acc.at[pl.ds(base + k * CHUNK, CHUNK)])
    plsc.subcore_barrier()

    def run(table_h, out_h):
        for blk in range(CPT // IBLK):
            pltpu.sync_copy(src_h.at[s].at[blk], sidx)
            pltpu.sync_copy(dst_h.at[s].at[blk], didx)
            for b in range(NBUF):
                pltpu.async_copy(table_h.at[sidx.at[b]], rows[b], gsem[b])

            def body(t, carry):
                j0 = t * NBUF
                for b in range(NBUF):
                    j = j0 + b
                    pltpu.make_async_copy(table_h.at[sidx.at[j]], rows[b],
                                          gsem[b]).wait()
                    pltpu.async_copy(rows[b], acc.at[didx.at[j]], ssem[b],
                                     add=True)
                for b in range(NBUF):
                    j = j0 + b

                    @pl.when(j + NBUF < IBLK)
                    def _():
                        pltpu.make_async_copy(rows[b], acc.at[didx.at[j]],
                                              ssem[b]).wait()
                        pltpu.async_copy(table_h.at[sidx.at[j + NBUF]],
                                         rows[b], gsem[b])

                return carry

            lax.fori_loop(0, IBLK // NBUF, body, 0)
            for b in range(NBUF):
                j = IBLK - NBUF + b
                pltpu.make_async_copy(rows[b], acc.at[didx.at[j]],
                                      ssem[b]).wait()
        plsc.subcore_barrier()
        for k in range(ROWS_PER_TILE // CHUNK):
            rb = rows[k % 2]
            sb = gsem[k % 2]
            if k >= 2:
                pltpu.make_async_copy(
                    rb, out_h.at[pl.ds(base + (k - 2) * CHUNK, CHUNK)],
                    sb).wait()
            pltpu.sync_copy(acc.at[pl.ds(base + k * CHUNK, CHUNK)], rb)
            pltpu.async_copy(rb, out_h.at[pl.ds(base + k * CHUNK, CHUNK)], sb)
        for k in (3, 4):
            pltpu.make_async_copy(
                rows[k % 2], out_h.at[pl.ds(base + k * CHUNK, CHUNK)],
                gsem[k % 2]).wait()

    @pl.when(c == 0)
    def _():
        run(ylo_h, alo_h)

    @pl.when(c == 1)
    def _():
        run(yhi_h, ahi_h)


_scat_kernel = functools.partial(
    pl.kernel,
    out_type=(jax.ShapeDtypeStruct((ACC_ROWS, H), jnp.float32),
              jax.ShapeDtypeStruct((ACC_ROWS, H), jnp.float32)),
    mesh=_mesh,
    scratch_types=[
        pltpu.VMEM_SHARED((ACC_ROWS, H), jnp.float32),
        pltpu.VMEM((IBLK, CHUNK), jnp.int32),
        pltpu.VMEM((IBLK, CHUNK), jnp.int32),
        pltpu.VMEM((CHUNK, H), jnp.float32),
        pltpu.VMEM((CHUNK, H), jnp.float32),
        pltpu.SemaphoreType.DMA,
        pltpu.SemaphoreType.DMA,
        pltpu.SemaphoreType.DMA,
        pltpu.SemaphoreType.DMA,
    ],
)(_scat_body)


RM = 1000
GRID = N_NODES // RM


def _dis_from(d0, d1):
    deg = d0[:, :1] + d1[:, :1] + 1.0
    return lax.rsqrt(deg)


def _mmT(a, w):
    return lax.dot_general(a, w, (((1,), (1,)), ((), ())),
                           preferred_element_type=jnp.float32)


def _tc1_body(x_ref, w0_ref, b0_ref, wc_ref, d0_ref, d1_ref, ylo_ref, yhi_ref):
    dis = _dis_from(d0_ref[...], d1_ref[...])
    h = jnp.maximum(_mmT(x_ref[...], w0_ref[...]) + b0_ref[...], 0.0)
    y = _mmT(h, wc_ref[...]) * dis
    ylo_ref[...] = y[:, :H]
    yhi_ref[...] = y[:, H:]


def _tc2_body(alo_ref, ahi_ref, ylo_ref, yhi_ref, b_ref, wc_ref,
              d0_ref, d1_ref, olo_ref, ohi_ref):
    dis = _dis_from(d0_ref[...], d1_ref[...])
    a = jnp.concatenate([alo_ref[...], ahi_ref[...]], axis=1)
    y = jnp.concatenate([ylo_ref[...], yhi_ref[...]], axis=1)
    h = jnp.maximum(dis * (a + y) + b_ref[...], 0.0)
    y2 = _mmT(h, wc_ref[...]) * dis
    olo_ref[...] = y2[:, :H]
    ohi_ref[...] = y2[:, H:]


def _tc3_body(alo_ref, ahi_ref, ylo_ref, yhi_ref, b_ref, w1_ref, b1_ref,
              d0_ref, d1_ref, out_ref):
    dis = _dis_from(d0_ref[...], d1_ref[...])
    a = jnp.concatenate([alo_ref[...], ahi_ref[...]], axis=1)
    y = jnp.concatenate([ylo_ref[...], yhi_ref[...]], axis=1)
    h = jnp.maximum(dis * (a + y) + b_ref[...], 0.0)
    out_ref[...] = _mmT(h, w1_ref[...]) + b1_ref[0, 0]


def _rows_spec(w):
    return pl.BlockSpec((RM, w), lambda i: (i, 0))


def _full_spec(shape):
    return pl.BlockSpec(shape, lambda i: tuple(0 for _ in shape))


_tc1 = pl.pallas_call(
    _tc1_body,
    grid=(GRID,),
    in_specs=[_rows_spec(D), _full_spec((D, D)), _full_spec((1, D)),
              _full_spec((D, D)), _rows_spec(H), _rows_spec(H)],
    out_specs=(_rows_spec(H), _rows_spec(H)),
    out_shape=(jax.ShapeDtypeStruct((N_NODES, H), jnp.float32),
               jax.ShapeDtypeStruct((N_NODES, H), jnp.float32)),
)

_tc2 = pl.pallas_call(
    _tc2_body,
    grid=(GRID,),
    in_specs=[_rows_spec(H), _rows_spec(H), _rows_spec(H), _rows_spec(H),
              _full_spec((1, D)), _full_spec((D, D)),
              _rows_spec(H), _rows_spec(H)],
    out_specs=(_rows_spec(H), _rows_spec(H)),
    out_shape=(jax.ShapeDtypeStruct((N_NODES, H), jnp.float32),
               jax.ShapeDtypeStruct((N_NODES, H), jnp.float32)),
)

_tc3 = pl.pallas_call(
    _tc3_body,
    grid=(GRID,),
    in_specs=[_rows_spec(H), _rows_spec(H), _rows_spec(H), _rows_spec(H),
              _full_spec((1, D)), _full_spec((H, D)), _full_spec((1, 1)),
              _rows_spec(H), _rows_spec(H)],
    out_specs=pl.BlockSpec((RM, H), lambda i: (i, 0)),
    out_shape=jax.ShapeDtypeStruct((N_NODES, H), jnp.float32),
)


@jax.jit
def kernel(x, edge_index, W0, b0, Wc1, bc1, Wc2, bc2, W1, b1):
    src = edge_index[0].astype(jnp.int32)
    dst = edge_index[1].astype(jnp.int32)
    pad = E_PAD - N_EDGES
    srcp = jnp.concatenate([src, jnp.zeros((pad,), jnp.int32)])
    dstp = jnp.concatenate([dst, jnp.full((pad,), N_NODES, jnp.int32)])
    src3 = srcp.reshape(NS, CPT // IBLK, IBLK, CHUNK)
    dst3 = dstp.reshape(NS, CPT // IBLK, IBLK, CHUNK)
    dst3w = dstp.reshape(NC * NS, CPW, CHUNK)
    ones_h = jnp.ones((CHUNK, H), jnp.float32)
    zz_h = jnp.zeros((CHUNK, H), jnp.float32)

    deg2 = _deg_kernel(dst3w, ones_h, zz_h)
    d0, d1 = deg2[0], deg2[1]

    b0r = b0.reshape(1, D)
    bc1r = bc1.reshape(1, D)
    bc2r = bc2.reshape(1, D)
    b1r = b1.reshape(1, 1)

    y1lo, y1hi = _tc1(x, W0, b0r, Wc1, d0[:N_NODES], d1[:N_NODES])
    a1lo, a1hi = _scat_kernel(src3, dst3, y1lo, y1hi, zz_h)
    W1p = jnp.zeros((H, D), jnp.float32).at[0].set(W1[0])
    y2lo, y2hi = _tc2(a1lo[:N_NODES], a1hi[:N_NODES], y1lo, y1hi,
                      bc1r, Wc2, d0[:N_NODES], d1[:N_NODES])
    a2lo, a2hi = _scat_kernel(src3, dst3, y2lo, y2hi, zz_h)
    outp = _tc3(a2lo[:N_NODES], a2hi[:N_NODES], y2lo, y2hi,
                bc2r, W1p, b1r, d0[:N_NODES], d1[:N_NODES])
    return outp[:, :1]


# speedup vs baseline: 7.3992x; 1.0225x over previous
import functools

import jax
import jax.numpy as jnp
from jax import lax
from jax.experimental import pallas as pl
from jax.experimental.pallas import tpu as pltpu
from jax.experimental.pallas import tpu_sc as plsc

N_NODES = 10000
N_EDGES = 160000
D = 256
H = 128
NC = 2
NS = 16
CHUNK = 128
ACC_ROWS = 10240
ROWS_PER_TILE = ACC_ROWS // NS
E_PAD = NS * 80 * CHUNK
CPT = E_PAD // NS // CHUNK
CPW = E_PAD // (NC * NS) // CHUNK

_mesh = plsc.VectorSubcoreMesh(core_axis_name="c", subcore_axis_name="s",
                               num_cores=NC, num_subcores=NS)


def _deg_body(dst_h, ones_h, zz_h, deg_h, dacc, dstv, onesv, rows):
    c = lax.axis_index("c")
    s = lax.axis_index("s")
    w = c * NS + s
    pltpu.sync_copy(dst_h.at[w], dstv)
    pltpu.sync_copy(ones_h, onesv)
    pltpu.sync_copy(zz_h, rows)
    base = s * ROWS_PER_TILE
    for k in range(ROWS_PER_TILE // CHUNK):
        pltpu.sync_copy(rows, dacc.at[pl.ds(base + k * CHUNK, CHUNK)])
    plsc.subcore_barrier()

    def chunk(j, carry):
        pltpu.sync_copy(onesv, dacc.at[dstv.at[j]], add=True)
        return carry

    lax.fori_loop(0, CPW, chunk, 0)
    plsc.subcore_barrier()
    for k in range(ROWS_PER_TILE // CHUNK):
        pltpu.sync_copy(dacc.at[pl.ds(base + k * CHUNK, CHUNK)], rows)
        pltpu.sync_copy(rows, deg_h.at[c].at[pl.ds(base + k * CHUNK, CHUNK)])


_deg_kernel = functools.partial(
    pl.kernel,
    out_type=jax.ShapeDtypeStruct((NC, ACC_ROWS, H), jnp.float32),
    mesh=_mesh,
    scratch_types=[
        pltpu.VMEM_SHARED((ACC_ROWS, H), jnp.float32),
        pltpu.VMEM((CPW, CHUNK), jnp.int32),
        pltpu.VMEM((CHUNK, H), jnp.float32),
        pltpu.VMEM((CHUNK, H), jnp.float32),
    ],
)(_deg_body)


NBUF = 2
IBLK = 40


def _scat_body(src_h, dst_h, ylo_h, yhi_h, zz_h, alo_h, ahi_h,
               acc, sidx, didx, r0, r1, g0, g1, s0, s1):
    rows = (r0, r1)
    gsem = (g0, g1)
    ssem = (s0, s1)
    c = lax.axis_index("c")
    s = lax.axis_index("s")
    pltpu.sync_copy(zz_h, r0)
    base = s * ROWS_PER_TILE
    for k in range(ROWS_PER_TILE // CHUNK):
        pltpu.sync_copy(r0, acc.at[pl.ds(base + k * CHUNK, CHUNK)])
    plsc.subcore_barrier()

    def run(table_h, out_h):
        for blk in range(CPT // IBLK):
            pltpu.sync_copy(src_h.at[s].at[blk], sidx)
            pltpu.sync_copy(dst_h.at[s].at[blk], didx)
            for b in range(NBUF):
                pltpu.async_copy(table_h.at[sidx.at[b]], rows[b], gsem[b])

            def body(t, carry):
                j0 = t * NBUF
                for b in range(NBUF):
                    j = j0 + b
                    pltpu.make_async_copy(table_h.at[sidx.at[j]], rows[b],
                                          gsem[b]).wait()
                    pltpu.async_copy(rows[b], acc.at[didx.at[j]], ssem[b],
                                     add=True)
                for b in range(NBUF):
                    j = j0 + b

                    @pl.when(j + NBUF < IBLK)
                    def _():
                        pltpu.make_async_copy(rows[b], acc.at[didx.at[j]],
                                              ssem[b]).wait()
                        pltpu.async_copy(table_h.at[sidx.at[j + NBUF]],
                                         rows[b], gsem[b])

                return carry

            lax.fori_loop(0, IBLK // NBUF, body, 0)
            for b in range(NBUF):
                j = IBLK - NBUF + b
                pltpu.make_async_copy(rows[b], acc.at[didx.at[j]],
                                      ssem[b]).wait()
        plsc.subcore_barrier()
        for k in range(ROWS_PER_TILE // CHUNK):
            rb = rows[k % 2]
            sb = gsem[k % 2]
            if k >= 2:
                pltpu.make_async_copy(
                    rb, out_h.at[pl.ds(base + (k - 2) * CHUNK, CHUNK)],
                    sb).wait()
            pltpu.sync_copy(acc.at[pl.ds(base + k * CHUNK, CHUNK)], rb)
            pltpu.async_copy(rb, out_h.at[pl.ds(base + k * CHUNK, CHUNK)], sb)
        for k in (3, 4):
            pltpu.make_async_copy(
                rows[k % 2], out_h.at[pl.ds(base + k * CHUNK, CHUNK)],
                gsem[k % 2]).wait()

    @pl.when(c == 0)
    def _():
        run(ylo_h, alo_h)

    @pl.when(c == 1)
    def _():
        run(yhi_h, ahi_h)


_scat_kernel = functools.partial(
    pl.kernel,
    out_type=(jax.ShapeDtypeStruct((ACC_ROWS, H), jnp.float32),
              jax.ShapeDtypeStruct((ACC_ROWS, H), jnp.float32)),
    mesh=_mesh,
    scratch_types=[
        pltpu.VMEM_SHARED((ACC_ROWS, H), jnp.float32),
        pltpu.VMEM((IBLK, CHUNK), jnp.int32),
        pltpu.VMEM((IBLK, CHUNK), jnp.int32),
        pltpu.VMEM((CHUNK, H), jnp.float32),
        pltpu.VMEM((CHUNK, H), jnp.float32),
        pltpu.SemaphoreType.DMA,
        pltpu.SemaphoreType.DMA,
        pltpu.SemaphoreType.DMA,
        pltpu.SemaphoreType.DMA,
    ],
)(_scat_body)


RM = 1000
GRID = N_NODES // RM


def _dis_from(d0, d1):
    deg = d0[:, :1] + d1[:, :1] + 1.0
    return lax.rsqrt(deg)


def _mmT(a, w):
    return lax.dot_general(a, w, (((1,), (1,)), ((), ())),
                           preferred_element_type=jnp.float32)


def _tc0_body(x_ref, w0_ref, b0_ref, h_ref):
    h_ref[...] = jnp.maximum(_mmT(x_ref[...], w0_ref[...]) + b0_ref[...], 0.0)


def _tc1_body(h_ref, wc_ref, d0_ref, d1_ref, ylo_ref, yhi_ref):
    dis = _dis_from(d0_ref[...], d1_ref[...])
    y = _mmT(h_ref[...], wc_ref[...]) * dis
    ylo_ref[...] = y[:, :H]
    yhi_ref[...] = y[:, H:]


def _tc2_body(alo_ref, ahi_ref, ylo_ref, yhi_ref, b_ref, wc_ref,
              d0_ref, d1_ref, olo_ref, ohi_ref):
    dis = _dis_from(d0_ref[...], d1_ref[...])
    a = jnp.concatenate([alo_ref[...], ahi_ref[...]], axis=1)
    y = jnp.concatenate([ylo_ref[...], yhi_ref[...]], axis=1)
    h = jnp.maximum(dis * (a + y) + b_ref[...], 0.0)
    y2 = _mmT(h, wc_ref[...]) * dis
    olo_ref[...] = y2[:, :H]
    ohi_ref[...] = y2[:, H:]


def _tc3_body(alo_ref, ahi_ref, ylo_ref, yhi_ref, b_ref, w1_ref, b1_ref,
              d0_ref, d1_ref, out_ref):
    dis = _dis_from(d0_ref[...], d1_ref[...])
    a = jnp.concatenate([alo_ref[...], ahi_ref[...]], axis=1)
    y = jnp.concatenate([ylo_ref[...], yhi_ref[...]], axis=1)
    h = jnp.maximum(dis * (a + y) + b_ref[...], 0.0)
    out_ref[...] = _mmT(h, w1_ref[...]) + b1_ref[0, 0]


def _rows_spec(w):
    return pl.BlockSpec((RM, w), lambda i: (i, 0))


def _full_spec(shape):
    return pl.BlockSpec(shape, lambda i: tuple(0 for _ in shape))


_tc0 = pl.pallas_call(
    _tc0_body,
    grid=(GRID,),
    in_specs=[_rows_spec(D), _full_spec((D, D)), _full_spec((1, D))],
    out_specs=_rows_spec(D),
    out_shape=jax.ShapeDtypeStruct((N_NODES, D), jnp.float32),
)

_tc1 = pl.pallas_call(
    _tc1_body,
    grid=(GRID,),
    in_specs=[_rows_spec(D), _full_spec((D, D)),
              _rows_spec(H), _rows_spec(H)],
    out_specs=(_rows_spec(H), _rows_spec(H)),
    out_shape=(jax.ShapeDtypeStruct((N_NODES, H), jnp.float32),
               jax.ShapeDtypeStruct((N_NODES, H), jnp.float32)),
)

_tc2 = pl.pallas_call(
    _tc2_body,
    grid=(GRID,),
    in_specs=[_rows_spec(H), _rows_spec(H), _rows_spec(H), _rows_spec(H),
              _full_spec((1, D)), _full_spec((D, D)),
              _rows_spec(H), _rows_spec(H)],
    out_specs=(_rows_spec(H), _rows_spec(H)),
    out_shape=(jax.ShapeDtypeStruct((N_NODES, H), jnp.float32),
               jax.ShapeDtypeStruct((N_NODES, H), jnp.float32)),
)

_tc3 = pl.pallas_call(
    _tc3_body,
    grid=(GRID,),
    in_specs=[_rows_spec(H), _rows_spec(H), _rows_spec(H), _rows_spec(H),
              _full_spec((1, D)), _full_spec((H, D)), _full_spec((1, 1)),
              _rows_spec(H), _rows_spec(H)],
    out_specs=pl.BlockSpec((RM, H), lambda i: (i, 0)),
    out_shape=jax.ShapeDtypeStruct((N_NODES, H), jnp.float32),
)


@jax.jit
def kernel(x, edge_index, W0, b0, Wc1, bc1, Wc2, bc2, W1, b1):
    src = edge_index[0].astype(jnp.int32)
    dst = edge_index[1].astype(jnp.int32)
    pad = E_PAD - N_EDGES
    srcp = jnp.concatenate([src, jnp.zeros((pad,), jnp.int32)])
    dstp = jnp.concatenate([dst, jnp.full((pad,), N_NODES, jnp.int32)])
    src3 = srcp.reshape(NS, CPT // IBLK, IBLK, CHUNK)
    dst3 = dstp.reshape(NS, CPT // IBLK, IBLK, CHUNK)
    dst3w = dstp.reshape(NC * NS, CPW, CHUNK)
    ones_h = jnp.ones((CHUNK, H), jnp.float32)
    zz_h = jnp.zeros((CHUNK, H), jnp.float32)

    deg2 = _deg_kernel(dst3w, ones_h, zz_h)
    d0, d1 = deg2[0], deg2[1]

    b0r = b0.reshape(1, D)
    bc1r = bc1.reshape(1, D)
    bc2r = bc2.reshape(1, D)
    b1r = b1.reshape(1, 1)

    h1 = _tc0(x, W0, b0r)
    y1lo, y1hi = _tc1(h1, Wc1, d0[:N_NODES], d1[:N_NODES])
    a1lo, a1hi = _scat_kernel(src3, dst3, y1lo, y1hi, zz_h)
    W1p = jnp.zeros((H, D), jnp.float32).at[0].set(W1[0])
    y2lo, y2hi = _tc2(a1lo[:N_NODES], a1hi[:N_NODES], y1lo, y1hi,
                      bc1r, Wc2, d0[:N_NODES], d1[:N_NODES])
    a2lo, a2hi = _scat_kernel(src3, dst3, y2lo, y2hi, zz_h)
    outp = _tc3(a2lo[:N_NODES], a2hi[:N_NODES], y2lo, y2hi,
                bc2r, W1p, b1r, d0[:N_NODES], d1[:N_NODES])
    return outp[:, :1]
